# baseline (speedup 1.0000x reference)
import functools

import jax
import jax.numpy as jnp
from jax import lax
from jax.experimental import pallas as pl
from jax.experimental.pallas import tpu as pltpu
from jax.experimental.pallas import tpu_sc as plsc

_NUM_CORES = 2
_NUM_SUBCORES = 16


def kernel(tokens, W_pos):
    B = tokens.shape[0]
    S = tokens.shape[1]
    D = W_pos.shape[1]
    nw = _NUM_CORES * _NUM_SUBCORES
    rows_per = S // nw

    mesh = plsc.VectorSubcoreMesh(
        core_axis_name="c", subcore_axis_name="s", num_cores=_NUM_CORES
    )

    @functools.partial(
        pl.kernel,
        mesh=mesh,
        out_type=jax.ShapeDtypeStruct((B, S, D), W_pos.dtype),
        scratch_types=[
            pltpu.VMEM((rows_per, D), W_pos.dtype),
            pltpu.SemaphoreType.DMA,
        ],
    )
    def sc_broadcast(w_hbm, out_hbm, rows_v, sem):
        wid = lax.axis_index("s") * _NUM_CORES + lax.axis_index("c")
        base = wid * rows_per
        pltpu.sync_copy(w_hbm.at[pl.ds(base, rows_per)], rows_v)
        copies = [
            pltpu.async_copy(rows_v, out_hbm.at[b, pl.ds(base, rows_per)], sem)
            for b in range(B)
        ]
        for c in copies:
            c.wait()

    return sc_broadcast(W_pos[:S])


# speedup vs baseline: 2.8533x; 2.8533x over previous
import jax
import jax.numpy as jnp
from jax.experimental import pallas as pl
from jax.experimental.pallas import tpu as pltpu

_CHUNKS = 4


def kernel(tokens, W_pos):
    B = tokens.shape[0]
    S = tokens.shape[1]
    D = W_pos.shape[1]
    rc = S // _CHUNKS

    def body(w_hbm, out_hbm, vmem, in_sem, out_sem):
        in_copies = [
            pltpu.make_async_copy(
                w_hbm.at[pl.ds(i * rc, rc)], vmem.at[pl.ds(i * rc, rc)], in_sem
            )
            for i in range(_CHUNKS)
        ]
        in_copies[0].start()
        out_copies = []
        for i in range(_CHUNKS):
            in_copies[i].wait()
            if i + 1 < _CHUNKS:
                in_copies[i + 1].start()
            for b in range(B):
                c = pltpu.async_copy(
                    vmem.at[pl.ds(i * rc, rc)],
                    out_hbm.at[b, pl.ds(i * rc, rc)],
                    out_sem,
                )
                out_copies.append(c)
        for c in out_copies:
            c.wait()

    return pl.pallas_call(
        body,
        in_specs=[pl.BlockSpec(memory_space=pltpu.MemorySpace.HBM)],
        out_specs=pl.BlockSpec(memory_space=pltpu.MemorySpace.HBM),
        out_shape=jax.ShapeDtypeStruct((B, S, D), W_pos.dtype),
        scratch_shapes=[
            pltpu.VMEM((S, D), W_pos.dtype),
            pltpu.SemaphoreType.DMA,
            pltpu.SemaphoreType.DMA,
        ],
    )(W_pos[:S])


# speedup vs baseline: 3.8947x; 1.3650x over previous
import jax
import jax.numpy as jnp
from jax.experimental import pallas as pl
from jax.experimental.pallas import tpu as pltpu

_CHUNKS = 1


def kernel(tokens, W_pos):
    B = tokens.shape[0]
    S = tokens.shape[1]
    D = W_pos.shape[1]
    rc = S // _CHUNKS

    def body(w_hbm, out_hbm, vmem, in_sem, out_sem):
        in_copies = [
            pltpu.make_async_copy(
                w_hbm.at[pl.ds(i * rc, rc)], vmem.at[pl.ds(i * rc, rc)], in_sem
            )
            for i in range(_CHUNKS)
        ]
        in_copies[0].start()
        out_copies = []
        for i in range(_CHUNKS):
            in_copies[i].wait()
            if i + 1 < _CHUNKS:
                in_copies[i + 1].start()
            for b in range(B):
                c = pltpu.async_copy(
                    vmem.at[pl.ds(i * rc, rc)],
                    out_hbm.at[b, pl.ds(i * rc, rc)],
                    out_sem,
                )
                out_copies.append(c)
        for c in out_copies:
            c.wait()

    return pl.pallas_call(
        body,
        in_specs=[pl.BlockSpec(memory_space=pltpu.MemorySpace.HBM)],
        out_specs=pl.BlockSpec(memory_space=pltpu.MemorySpace.HBM),
        out_shape=jax.ShapeDtypeStruct((B, S, D), W_pos.dtype),
        scratch_shapes=[
            pltpu.VMEM((S, D), W_pos.dtype),
            pltpu.SemaphoreType.DMA,
            pltpu.SemaphoreType.DMA,
        ],
    )(W_pos[:S])
